# baseline (speedup 1.0000x reference)
import functools
import jax
import jax.numpy as jnp
from jax import lax
from jax.experimental import pallas as pl
from jax.experimental.pallas import tpu as pltpu
from jax.experimental.pallas import tpu_sc as plsc

_N = 10000
_E = 320000
_D = 128
_F = 256
_B = 8
_NE = 16
_FE = 256
_NPF = 48
_FN = _B * _NPF
_FEE = _B * _FE

_NC = 2
_NS = 16
_CH = 80
_EPW = _E // (_NC * _NS)
_NCHUNK = _EPW // _CH
_WB = 80
_NWB = _N // _WB
_WBPT = (_NWB + _NS - 1) // _NS

_f32 = jnp.float32


_CA = 128
_NCA = (_E // _NC) // _CA
_ZC = 80
_NZ = _N // _ZC


def _gat_body(feat, srci, dsti, z8, o8,
              gout, cnt_out,
              cacc, srcv, dstv, rows, onev, zbuf, sem):
    c = lax.axis_index("c")
    s = lax.axis_index("s")
    w = c * _NS + s
    my0 = s * _N

    pltpu.sync_copy(z8, zbuf)

    @pl.loop(0, _NZ)
    def _zero(j):
        pltpu.sync_copy(zbuf, cacc.at[pl.ds(my0 + j * _ZC, _ZC)])

    pltpu.sync_copy(o8, onev)

    @pl.loop(0, _NCHUNK)
    def _chunk(i):
        base = pl.multiple_of(w * _EPW + i * _CH, _CH)
        pltpu.sync_copy(srci.at[pl.ds(base, _CH)], srcv)
        pltpu.async_copy(feat.at[srcv], rows, sem).wait()
        for g in range(_NS):
            pltpu.sync_copy(rows.at[:, pl.ds(8 * g, 8)],
                            gout.at[pl.ds(g * _E + base, _CH)])
        pltpu.sync_copy(dsti.at[pl.ds(base, _CH)], dstv)
        for k in range(_CH // 16):
            dstv[pl.ds(k * 16, 16)] = dstv[pl.ds(k * 16, 16)] + my0
        pltpu.sync_copy(onev, cacc.at[dstv], add=True)

    @pl.loop(0, _NZ)
    def _wb(j):
        r0 = my0 + j * _ZC
        pltpu.sync_copy(cacc.at[pl.ds(r0, _ZC)], zbuf)
        pltpu.sync_copy(zbuf, cnt_out.at[pl.ds(c * _NS * _N + r0, _ZC)])


def _make_gat_call():
    mesh = plsc.VectorSubcoreMesh(core_axis_name="c", subcore_axis_name="s")
    return pl.kernel(
        _gat_body,
        out_type=[
            jax.ShapeDtypeStruct((_NS * _E, 8), _f32),
            jax.ShapeDtypeStruct((_NC * _NS * _N, 8), _f32),
        ],
        mesh=mesh,
        scratch_types=[
            pltpu.VMEM_SHARED((_NS * _N, 8), _f32),
            pltpu.VMEM((_CH,), jnp.int32),
            pltpu.VMEM((_CH,), jnp.int32),
            pltpu.VMEM((_CH, _D), _f32),
            pltpu.VMEM((_CH, 8), _f32),
            pltpu.VMEM((_ZC, 8), _f32),
            pltpu.SemaphoreType.DMA,
        ],
        compiler_params=pltpu.CompilerParams(use_tc_tiling_on_sc=False),
    )


def _sca_body(gout, dsti, z8,
              agg_out,
              acc, dstv, rows, zbuf, sem):
    c = lax.axis_index("c")
    s = lax.axis_index("s")
    my0 = s * _N

    pltpu.sync_copy(z8, zbuf)

    @pl.loop(0, _NZ)
    def _zero(j):
        pltpu.sync_copy(zbuf, acc.at[pl.ds(my0 + j * _ZC, _ZC)])

    ebase = c * (_E // _NC)

    @pl.loop(0, _NCA)
    def _chunk(i):
        base = pl.multiple_of(ebase + i * _CA, _CA)
        pltpu.sync_copy(dsti.at[pl.ds(base, _CA)], dstv)
        for k in range(_CA // 16):
            dstv[pl.ds(k * 16, 16)] = dstv[pl.ds(k * 16, 16)] + my0
        pltpu.sync_copy(gout.at[pl.ds(s * _E + base, _CA)], rows)
        pltpu.sync_copy(rows, acc.at[dstv], add=True)

    @pl.loop(0, _NZ)
    def _wb(j):
        r0 = my0 + j * _ZC
        pltpu.sync_copy(acc.at[pl.ds(r0, _ZC)], zbuf)
        pltpu.sync_copy(zbuf, agg_out.at[pl.ds(c * _NS * _N + r0, _ZC)])


def _make_sca_call():
    mesh = plsc.VectorSubcoreMesh(core_axis_name="c", subcore_axis_name="s")
    return pl.kernel(
        _sca_body,
        out_type=jax.ShapeDtypeStruct((_NC * _NS * _N, 8), _f32),
        mesh=mesh,
        scratch_types=[
            pltpu.VMEM_SHARED((_NS * _N, 8), _f32),
            pltpu.VMEM((_CA,), jnp.int32),
            pltpu.VMEM((_CA, 8), _f32),
            pltpu.VMEM((_ZC, 8), _f32),
            pltpu.SemaphoreType.DMA,
        ],
        compiler_params=pltpu.CompilerParams(use_tc_tiling_on_sc=False),
    )


def _dense_tc_body(a0, a1, ct, x, wl, bl, wr, o):
    ones = jnp.full((_NC * _NS * 8, 1), 0.125, _f32)
    cnt = jnp.maximum(jnp.dot(ct[...], ones, preferred_element_type=_f32), 1.0)
    mean = (a0[...] + a1[...]) / cnt
    acc = jnp.dot(mean, wl[...], preferred_element_type=_f32)
    acc += jnp.dot(x[...], wr[...], preferred_element_type=_f32)
    o[...] = jnp.maximum(acc + bl[...], 0.0)


def _dense_tc(a0, a1, ct, x, wl, bl, wr):
    R = 1000
    grid = (_N // R,)
    row = lambda i: (i, 0)
    return pl.pallas_call(
        _dense_tc_body,
        grid=grid,
        in_specs=[
            pl.BlockSpec((R, _D), row),
            pl.BlockSpec((R, _D), row),
            pl.BlockSpec((R, _NC * _NS * 8), row),
            pl.BlockSpec((R, _D), row),
            pl.BlockSpec((_D, _D), lambda i: (0, 0)),
            pl.BlockSpec((1, _D), lambda i: (0, 0)),
            pl.BlockSpec((_D, _D), lambda i: (0, 0)),
        ],
        out_specs=pl.BlockSpec((R, _D), row),
        out_shape=jax.ShapeDtypeStruct((_N, _D), _f32),
    )(a0, a1, ct, x, wl, bl, wr)


def _pool_body(h, seg, extids, emb, z128, z16, o16,
               psum_out, pcnt_out, ext_out,
               psum, pcnt, segv, rows, onev, sbuf, cbuf, xbuf, cacc,
               idv, erows, sem):
    c = lax.axis_index("c")
    s = lax.axis_index("s")

    @pl.when(c == 0)
    def _zero():
        pltpu.sync_copy(z128.at[pl.ds(0, 64)], rows.at[pl.ds(0, 64)])
        for j in range(_F // 64):
            pltpu.sync_copy(rows.at[pl.ds(0, 64)],
                            psum.at[pl.ds(s * _F + j * 64, 64)])
        pltpu.sync_copy(z16.at[pl.ds(0, 64)], onev.at[pl.ds(0, 64)])
        for j in range(_F // 64):
            pltpu.sync_copy(onev.at[pl.ds(0, 64)],
                            pcnt.at[pl.ds(s * _F + j * 64, 64)])
        pltpu.sync_copy(o16, onev)

    @pl.when(c == 1)
    def _emb():
        pltpu.sync_copy(extids.at[pl.ds(s * 8, 8)], idv)
        pltpu.async_copy(emb.at[idv], erows, sem).wait()
        pltpu.sync_copy(erows, ext_out.at[pl.ds(s * 8, 8)])

    plsc.subcore_barrier()

    @pl.when(c == 0)
    def _pool():
        @pl.loop(0, (_NCHUNK + _NS - 1) // _NS)
        def _j(j):
            chunk = s + _NS * j

            @pl.when(chunk < _NCHUNK)
            def _do():
                base = pl.multiple_of(chunk * _CH, _CH)
                pltpu.sync_copy(seg.at[pl.ds(base, _CH)], segv)
                for k in range(_CH // 16):
                    segv[pl.ds(k * 16, 16)] = (
                        segv[pl.ds(k * 16, 16)] + s * _F)
                pltpu.sync_copy(h.at[pl.ds(base, _CH)], rows)
                pltpu.sync_copy(rows, psum.at[segv], add=True)
                pltpu.sync_copy(onev, pcnt.at[segv], add=True)

    plsc.subcore_barrier()

    @pl.when(c == 0)
    def _wb():
        pltpu.sync_copy(psum.at[pl.ds(s * 16, 16)], xbuf)
        pltpu.sync_copy(pcnt.at[pl.ds(s * 16, 16)], cacc)

        @pl.loop(1, _NS)
        def _t(t):
            pltpu.sync_copy(psum.at[pl.ds(t * _F + s * 16, 16)], sbuf)
            pltpu.sync_copy(pcnt.at[pl.ds(t * _F + s * 16, 16)], cbuf)
            for r in range(16):
                for k in range(_D // 16):
                    xbuf[r, pl.ds(k * 16, 16)] = (
                        xbuf[r, pl.ds(k * 16, 16)]
                        + sbuf[r, pl.ds(k * 16, 16)])
                cacc[r, :] = cacc[r, :] + cbuf[r, :]

        pltpu.sync_copy(xbuf, psum_out.at[pl.ds(s * 16, 16)])
        pltpu.sync_copy(cacc, pcnt_out.at[pl.ds(s * 16, 16)])


def _make_pool_call():
    mesh = plsc.VectorSubcoreMesh(core_axis_name="c", subcore_axis_name="s")
    return pl.kernel(
        _pool_body,
        out_type=[
            jax.ShapeDtypeStruct((_F, _D), _f32),
            jax.ShapeDtypeStruct((_F, 16), _f32),
            jax.ShapeDtypeStruct((_B * _NE, _D), _f32),
        ],
        mesh=mesh,
        scratch_types=[
            pltpu.VMEM_SHARED((_NS * _F, _D), _f32),
            pltpu.VMEM_SHARED((_NS * _F, 16), _f32),
            pltpu.VMEM((_CH,), jnp.int32),
            pltpu.VMEM((_CH, _D), _f32),
            pltpu.VMEM((_CH, 16), _f32),
            pltpu.VMEM((16, _D), _f32),
            pltpu.VMEM((16, 16), _f32),
            pltpu.VMEM((16, _D), _f32),
            pltpu.VMEM((16, 16), _f32),
            pltpu.VMEM((8,), jnp.int32),
            pltpu.VMEM((8, _D), _f32),
            pltpu.SemaphoreType.DMA,
        ],
    )


def _fcg_body(psum, pcnt, xext, fe, w1l, b1l, w1r, w2l, b2l, w2r,
              p1w, p1b, p2w, p2b, p3w, p3b, o):
    xcfg = psum[...] / jnp.maximum(pcnt[:, 0:1], 1.0)
    r_b = lax.broadcasted_iota(jnp.int32, (_FN, _F), 0) // _NPF
    r_i = lax.broadcasted_iota(jnp.int32, (_FN, _F), 0) % _NPF
    q = lax.broadcasted_iota(jnp.int32, (_FN, _F), 1)
    A1 = jnp.where((r_i < _F // _B) & (q == r_b * (_F // _B) + r_i), 1.0, 0.0)
    r_b2 = lax.broadcasted_iota(jnp.int32, (_FN, _B * _NE), 0) // _NPF
    r_i2 = lax.broadcasted_iota(jnp.int32, (_FN, _B * _NE), 0) % _NPF
    p = lax.broadcasted_iota(jnp.int32, (_FN, _B * _NE), 1)
    A2 = jnp.where((r_i2 >= _F // _B) & (p == r_b2 * _NE + r_i2 - _F // _B),
                   1.0, 0.0)
    xt = (jnp.dot(A1, xcfg, preferred_element_type=_f32)
          + jnp.dot(A2, xext[...], preferred_element_type=_f32))

    fs = fe[:, 0:1]
    fd = fe[:, 1:2]
    node_iota = lax.broadcasted_iota(jnp.int32, (_FEE, _FN), 1)
    S = (node_iota == fs).astype(_f32)
    Dm = (node_iota == fd).astype(_f32)
    ones_col = jnp.ones((_FEE, 1), _f32)
    cnt = lax.dot_general(Dm, ones_col, (((0,), (0,)), ((), ())),
                          preferred_element_type=_f32)
    cnt = jnp.maximum(cnt, 1.0)

    def sage(xin, wl, bl, wr):
        gath = jnp.dot(S, xin, preferred_element_type=_f32)
        sums = lax.dot_general(Dm, gath, (((0,), (0,)), ((), ())),
                               preferred_element_type=_f32)
        mean = sums / cnt
        out = jnp.dot(mean, wl[...], preferred_element_type=_f32)
        out += jnp.dot(xin, wr[...], preferred_element_type=_f32)
        return jnp.maximum(out + bl[...], 0.0)

    g = sage(xt, w1l, b1l, w1r)
    g = sage(g, w2l, b2l, w2r)

    bin_of = lax.broadcasted_iota(jnp.int32, (_B, _FN), 1) // _NPF
    bid = lax.broadcasted_iota(jnp.int32, (_B, _FN), 0)
    P = jnp.where(bin_of == bid, 1.0 / _NPF, 0.0)
    pooled = jnp.dot(P, g, preferred_element_type=_f32)

    z = jnp.dot(pooled, p1w[...], preferred_element_type=_f32) + p1b[...]
    z = jnp.dot(z, p2w[...], preferred_element_type=_f32) + p2b[...]
    z = jnp.dot(z, p3w[...], preferred_element_type=_f32) + p3b[...]
    o[...] = jax.nn.sigmoid(z)


def _fcg_tc(psum, pcnt, xext, fe, w1l, b1l, w1r, w2l, b2l, w2r,
            p1w, p1b, p2w, p2b, p3w, p3b):
    return pl.pallas_call(
        _fcg_body,
        out_shape=jax.ShapeDtypeStruct((_B, 1), _f32),
    )(psum, pcnt, xext, fe, w1l, b1l, w1r, w2l, b2l, w2r,
      p1w, p1b, p2w, p2b, p3w, p3b)


@jax.jit
def kernel(x, edge_index, cfg_batch, ext_names, func_edges,
           cfg1_Wl, cfg1_bl, cfg1_Wr, cfg2_Wl, cfg2_bl, cfg2_Wr,
           fcg1_Wl, fcg1_bl, fcg1_Wr, fcg2_Wl, fcg2_bl, fcg2_Wr,
           emb, pj1_W, pj1_b, pj2_W, pj2_b, pj3_W, pj3_b):
    src = edge_index[0]
    dst = edge_index[1]
    z128 = jnp.zeros((_WB, _D), _f32)
    z16 = jnp.zeros((_WB, 16), _f32)
    o16 = jnp.ones((_CH, 16), _f32)
    z8 = jnp.zeros((_ZC, 8), _f32)
    o8 = jnp.ones((_CH, 8), _f32)

    gat_call = _make_gat_call()
    sca_call = _make_sca_call()
    pool_call = _make_pool_call()

    def _ungroup(aggT):
        u = aggT.reshape(_NC, _NS, _N, 8).transpose(0, 2, 1, 3)
        u = u.reshape(_NC, _N, _D)
        return u[0], u[1]

    g1, cnt3 = gat_call(x, src, dst, z8, o8)
    ct = cnt3.reshape(_NC * _NS, _N, 8).transpose(1, 0, 2)
    ct = ct.reshape(_N, _NC * _NS * 8)
    a0, a1 = _ungroup(sca_call(g1, dst, z8))
    h = _dense_tc(a0, a1, ct, x, cfg1_Wl, cfg1_bl.reshape(1, _D), cfg1_Wr)
    g2, _ = gat_call(h, src, dst, z8, o8)
    b0, b1 = _ungroup(sca_call(g2, dst, z8))
    h2 = _dense_tc(b0, b1, ct, h, cfg2_Wl, cfg2_bl.reshape(1, _D), cfg2_Wr)

    psum, pcnt, xext = pool_call(h2, cfg_batch, ext_names.reshape(-1), emb,
                                 z128, z16, o16)

    off = (jnp.arange(_B, dtype=func_edges.dtype) * _NPF)[:, None, None]
    fe = (func_edges + off).transpose(1, 0, 2).reshape(2, _FEE).T
    fe = fe.astype(jnp.int32)

    return _fcg_tc(psum, pcnt, xext, fe,
                   fcg1_Wl, fcg1_bl.reshape(1, _D), fcg1_Wr,
                   fcg2_Wl, fcg2_bl.reshape(1, _D), fcg2_Wr,
                   pj1_W, pj1_b.reshape(1, -1), pj2_W, pj2_b.reshape(1, -1),
                   pj3_W, pj3_b.reshape(1, 1))


# speedup vs baseline: 1.7464x; 1.7464x over previous
import functools
import jax
import jax.numpy as jnp
from jax import lax
from jax.experimental import pallas as pl
from jax.experimental.pallas import tpu as pltpu
from jax.experimental.pallas import tpu_sc as plsc

_N = 10000
_E = 320000
_D = 128
_F = 256
_B = 8
_NE = 16
_FE = 256
_NPF = 48
_FN = _B * _NPF
_FEE = _B * _FE

_NC = 2
_NS = 16
_CH = 80
_EPW = _E // (_NC * _NS)
_NCHUNK = _EPW // _CH
_WB = 80
_NWB = _N // _WB
_WBPT = (_NWB + _NS - 1) // _NS

_f32 = jnp.float32


_CA = 128
_NCA = (_E // _NC) // _CA
_ZC = 80
_NZ = _N // _ZC


def _gat_core(with_cnt, feat, srci, dshift, z8, o8,
              gout, cnt_out,
              cacc, srcv0, srcv1, dstv, rows0, rows1, onev, zbuf,
              semg0, semg1, semw0, semw1, semc):
    c = lax.axis_index("c")
    s = lax.axis_index("s")
    w = c * _NS + s
    my0 = s * _N
    ebase = w * _EPW
    srcvs, rowss, semgs, semws = (srcv0, srcv1), (rows0, rows1), \
        (semg0, semg1), (semw0, semw1)

    if with_cnt:
        pltpu.sync_copy(z8, zbuf)

        @pl.loop(0, _NZ)
        def _zero(j):
            pltpu.sync_copy(zbuf, cacc.at[pl.ds(my0 + j * _ZC, _ZC)])

        pltpu.sync_copy(o8, onev)

    pltpu.sync_copy(srci.at[pl.ds(ebase, _CH)], srcv0)
    pltpu.async_copy(feat.at[srcv0], rows0, semg0)

    @pl.loop(0, _NCHUNK)
    def _chunk(i):
        base = pl.multiple_of(ebase + i * _CH, _CH)

        def _steps(b):
            sv, rw, sg, swr = srcvs[b], rowss[b], semgs[b], semws[b]
            pltpu.make_async_copy(feat.at[sv], rw, sg).wait()

            @pl.when(i + 1 < _NCHUNK)
            def _pf():
                nb = pl.multiple_of(base + _CH, _CH)
                osv, orw = srcvs[1 - b], rowss[1 - b]

                @pl.when(i >= 1)
                def _dw():
                    for g in range(_NS):
                        pltpu.make_async_copy(
                            orw.at[:, pl.ds(8 * g, 8)],
                            gout.at[pl.ds(g * _E + nb, _CH)],
                            semws[1 - b]).wait()

                pltpu.sync_copy(srci.at[pl.ds(nb, _CH)], osv)
                pltpu.async_copy(feat.at[osv], orw, semgs[1 - b])

            for g in range(_NS):
                pltpu.async_copy(rw.at[:, pl.ds(8 * g, 8)],
                                 gout.at[pl.ds(g * _E + base, _CH)], swr)

            if with_cnt:
                pltpu.sync_copy(dshift.at[pl.ds(s * _E + base, _CH)], dstv)
                pltpu.async_copy(onev, cacc.at[dstv], semc, add=True).wait()

        @pl.when(i % 2 == 0)
        def _b0():
            _steps(0)

        @pl.when(i % 2 == 1)
        def _b1():
            _steps(1)

    for b in range(2):
        last = ebase
        for g in range(_NS):
            pltpu.make_async_copy(rowss[b].at[:, pl.ds(8 * g, 8)],
                                  gout.at[pl.ds(g * _E + last, _CH)],
                                  semws[b]).wait()

    if with_cnt:
        @pl.loop(0, _NZ)
        def _wb(j):
            r0 = my0 + j * _ZC
            pltpu.sync_copy(cacc.at[pl.ds(r0, _ZC)], zbuf)
            pltpu.sync_copy(zbuf, cnt_out.at[pl.ds(c * _NS * _N + r0, _ZC)])


def _make_gat_call(with_cnt):
    mesh = plsc.VectorSubcoreMesh(core_axis_name="c", subcore_axis_name="s")
    if with_cnt:
        out_type = [
            jax.ShapeDtypeStruct((_NS * _E, 8), _f32),
            jax.ShapeDtypeStruct((_NC * _NS * _N, 8), _f32),
        ]

        def body(feat, srci, dshift, z8, o8, gout, cnt_out, *scr):
            _gat_core(True, feat, srci, dshift, z8, o8, gout, cnt_out, *scr)
    else:
        out_type = jax.ShapeDtypeStruct((_NS * _E, 8), _f32)

        def body(feat, srci, dshift, z8, o8, gout, *scr):
            _gat_core(False, feat, srci, dshift, z8, o8, gout, None, *scr)

    return pl.kernel(
        body,
        out_type=out_type,
        mesh=mesh,
        scratch_types=[
            pltpu.VMEM_SHARED((_NS * _N, 8), _f32),
            pltpu.VMEM((_CH,), jnp.int32),
            pltpu.VMEM((_CH,), jnp.int32),
            pltpu.VMEM((_CH,), jnp.int32),
            pltpu.VMEM((_CH, _D), _f32),
            pltpu.VMEM((_CH, _D), _f32),
            pltpu.VMEM((_CH, 8), _f32),
            pltpu.VMEM((_ZC, 8), _f32),
            pltpu.SemaphoreType.DMA,
            pltpu.SemaphoreType.DMA,
            pltpu.SemaphoreType.DMA,
            pltpu.SemaphoreType.DMA,
            pltpu.SemaphoreType.DMA,
        ],
        compiler_params=pltpu.CompilerParams(use_tc_tiling_on_sc=False),
    )


def _sca_body(gout, dshift, z8,
              agg_out,
              acc, dstv0, dstv1, rows0, rows1, zbuf,
              seml0, seml1, sems0, sems1):
    c = lax.axis_index("c")
    s = lax.axis_index("s")
    my0 = s * _N
    dstvs, rowss = (dstv0, dstv1), (rows0, rows1)
    semls, semss = (seml0, seml1), (sems0, sems1)

    pltpu.sync_copy(z8, zbuf)

    @pl.loop(0, _NZ)
    def _zero(j):
        pltpu.sync_copy(zbuf, acc.at[pl.ds(my0 + j * _ZC, _ZC)])

    ebase = c * (_E // _NC)
    ibase = s * _E + ebase

    def _loads(i, b):
        off = pl.multiple_of(i * _CA, _CA)
        pltpu.async_copy(dshift.at[pl.ds(ibase + off, _CA)], dstvs[b],
                         semls[b])
        pltpu.async_copy(gout.at[pl.ds(ibase + off, _CA)], rowss[b],
                         semls[b])

    _loads(0, 0)
    _loads(1, 1)

    @pl.loop(0, _NCA // 2)
    def _grp(g):
        i0 = g * 2
        for b in range(2):
            i = i0 + b
            off = pl.multiple_of(i * _CA, _CA)
            pltpu.make_async_copy(dshift.at[pl.ds(ibase + off, _CA)],
                                  dstvs[b], semls[b]).wait()
            pltpu.make_async_copy(gout.at[pl.ds(ibase + off, _CA)],
                                  rowss[b], semls[b]).wait()
            pltpu.async_copy(rowss[b], acc.at[dstvs[b]], semss[b], add=True)
        for b in range(2):
            pltpu.make_async_copy(rowss[b], acc.at[dstvs[b]],
                                  semss[b]).wait()

            @pl.when(g + 1 < _NCA // 2)
            def _rf():
                _loads(i0 + 2 + b, b)

    @pl.loop(0, _NZ)
    def _wb(j):
        r0 = my0 + j * _ZC
        pltpu.sync_copy(acc.at[pl.ds(r0, _ZC)], zbuf)
        pltpu.sync_copy(zbuf, agg_out.at[pl.ds(c * _NS * _N + r0, _ZC)])


def _make_sca_call():
    mesh = plsc.VectorSubcoreMesh(core_axis_name="c", subcore_axis_name="s")
    return pl.kernel(
        _sca_body,
        out_type=jax.ShapeDtypeStruct((_NC * _NS * _N, 8), _f32),
        mesh=mesh,
        scratch_types=[
            pltpu.VMEM_SHARED((_NS * _N, 8), _f32),
            pltpu.VMEM((_CA,), jnp.int32),
            pltpu.VMEM((_CA,), jnp.int32),
            pltpu.VMEM((_CA, 8), _f32),
            pltpu.VMEM((_CA, 8), _f32),
            pltpu.VMEM((_ZC, 8), _f32),
            pltpu.SemaphoreType.DMA,
            pltpu.SemaphoreType.DMA,
            pltpu.SemaphoreType.DMA,
            pltpu.SemaphoreType.DMA,
        ],
        compiler_params=pltpu.CompilerParams(use_tc_tiling_on_sc=False),
    )


def _dense_tc_body(a0, a1, ct, x, wl, bl, wr, o):
    ones = jnp.full((_NC * _NS * 8, 1), 0.125, _f32)
    cnt = jnp.maximum(jnp.dot(ct[...], ones, preferred_element_type=_f32), 1.0)
    mean = (a0[...] + a1[...]) / cnt
    acc = jnp.dot(mean, wl[...], preferred_element_type=_f32)
    acc += jnp.dot(x[...], wr[...], preferred_element_type=_f32)
    o[...] = jnp.maximum(acc + bl[...], 0.0)


def _dense_tc(a0, a1, ct, x, wl, bl, wr):
    R = 1000
    grid = (_N // R,)
    row = lambda i: (i, 0)
    return pl.pallas_call(
        _dense_tc_body,
        grid=grid,
        in_specs=[
            pl.BlockSpec((R, _D), row),
            pl.BlockSpec((R, _D), row),
            pl.BlockSpec((R, _NC * _NS * 8), row),
            pl.BlockSpec((R, _D), row),
            pl.BlockSpec((_D, _D), lambda i: (0, 0)),
            pl.BlockSpec((1, _D), lambda i: (0, 0)),
            pl.BlockSpec((_D, _D), lambda i: (0, 0)),
        ],
        out_specs=pl.BlockSpec((R, _D), row),
        out_shape=jax.ShapeDtypeStruct((_N, _D), _f32),
    )(a0, a1, ct, x, wl, bl, wr)


def _pool_body(h, seg, extids, emb, z128, z16, o16,
               psum_out, pcnt_out, ext_out,
               psum, pcnt, segv, rows, onev, sbuf, cbuf, xbuf, cacc,
               idv, erows, sem):
    c = lax.axis_index("c")
    s = lax.axis_index("s")

    @pl.when(c == 0)
    def _zero():
        pltpu.sync_copy(z128.at[pl.ds(0, 64)], rows.at[pl.ds(0, 64)])
        for j in range(_F // 64):
            pltpu.sync_copy(rows.at[pl.ds(0, 64)],
                            psum.at[pl.ds(s * _F + j * 64, 64)])
        pltpu.sync_copy(z16.at[pl.ds(0, 64)], onev.at[pl.ds(0, 64)])
        for j in range(_F // 64):
            pltpu.sync_copy(onev.at[pl.ds(0, 64)],
                            pcnt.at[pl.ds(s * _F + j * 64, 64)])
        pltpu.sync_copy(o16, onev)

    @pl.when(c == 1)
    def _emb():
        pltpu.sync_copy(extids.at[pl.ds(s * 8, 8)], idv)
        pltpu.async_copy(emb.at[idv], erows, sem).wait()
        pltpu.sync_copy(erows, ext_out.at[pl.ds(s * 8, 8)])

    plsc.subcore_barrier()

    @pl.when(c == 0)
    def _pool():
        @pl.loop(0, (_NCHUNK + _NS - 1) // _NS)
        def _j(j):
            chunk = s + _NS * j

            @pl.when(chunk < _NCHUNK)
            def _do():
                base = pl.multiple_of(chunk * _CH, _CH)
                pltpu.sync_copy(seg.at[pl.ds(base, _CH)], segv)
                for k in range(_CH // 16):
                    segv[pl.ds(k * 16, 16)] = (
                        segv[pl.ds(k * 16, 16)] + s * _F)
                pltpu.sync_copy(h.at[pl.ds(base, _CH)], rows)
                pltpu.sync_copy(rows, psum.at[segv], add=True)
                pltpu.sync_copy(onev, pcnt.at[segv], add=True)

    plsc.subcore_barrier()

    @pl.when(c == 0)
    def _wb():
        pltpu.sync_copy(psum.at[pl.ds(s * 16, 16)], xbuf)
        pltpu.sync_copy(pcnt.at[pl.ds(s * 16, 16)], cacc)

        @pl.loop(1, _NS)
        def _t(t):
            pltpu.sync_copy(psum.at[pl.ds(t * _F + s * 16, 16)], sbuf)
            pltpu.sync_copy(pcnt.at[pl.ds(t * _F + s * 16, 16)], cbuf)
            for r in range(16):
                for k in range(_D // 16):
                    xbuf[r, pl.ds(k * 16, 16)] = (
                        xbuf[r, pl.ds(k * 16, 16)]
                        + sbuf[r, pl.ds(k * 16, 16)])
                cacc[r, :] = cacc[r, :] + cbuf[r, :]

        pltpu.sync_copy(xbuf, psum_out.at[pl.ds(s * 16, 16)])
        pltpu.sync_copy(cacc, pcnt_out.at[pl.ds(s * 16, 16)])


def _make_pool_call():
    mesh = plsc.VectorSubcoreMesh(core_axis_name="c", subcore_axis_name="s")
    return pl.kernel(
        _pool_body,
        out_type=[
            jax.ShapeDtypeStruct((_F, _D), _f32),
            jax.ShapeDtypeStruct((_F, 16), _f32),
            jax.ShapeDtypeStruct((_B * _NE, _D), _f32),
        ],
        mesh=mesh,
        scratch_types=[
            pltpu.VMEM_SHARED((_NS * _F, _D), _f32),
            pltpu.VMEM_SHARED((_NS * _F, 16), _f32),
            pltpu.VMEM((_CH,), jnp.int32),
            pltpu.VMEM((_CH, _D), _f32),
            pltpu.VMEM((_CH, 16), _f32),
            pltpu.VMEM((16, _D), _f32),
            pltpu.VMEM((16, 16), _f32),
            pltpu.VMEM((16, _D), _f32),
            pltpu.VMEM((16, 16), _f32),
            pltpu.VMEM((8,), jnp.int32),
            pltpu.VMEM((8, _D), _f32),
            pltpu.SemaphoreType.DMA,
        ],
    )


def _fcg_body(psum, pcnt, xext, fe, w1l, b1l, w1r, w2l, b2l, w2r,
              p1w, p1b, p2w, p2b, p3w, p3b, o):
    xcfg = psum[...] / jnp.maximum(pcnt[:, 0:1], 1.0)
    r_b = lax.broadcasted_iota(jnp.int32, (_FN, _F), 0) // _NPF
    r_i = lax.broadcasted_iota(jnp.int32, (_FN, _F), 0) % _NPF
    q = lax.broadcasted_iota(jnp.int32, (_FN, _F), 1)
    A1 = jnp.where((r_i < _F // _B) & (q == r_b * (_F // _B) + r_i), 1.0, 0.0)
    r_b2 = lax.broadcasted_iota(jnp.int32, (_FN, _B * _NE), 0) // _NPF
    r_i2 = lax.broadcasted_iota(jnp.int32, (_FN, _B * _NE), 0) % _NPF
    p = lax.broadcasted_iota(jnp.int32, (_FN, _B * _NE), 1)
    A2 = jnp.where((r_i2 >= _F // _B) & (p == r_b2 * _NE + r_i2 - _F // _B),
                   1.0, 0.0)
    xt = (jnp.dot(A1, xcfg, preferred_element_type=_f32)
          + jnp.dot(A2, xext[...], preferred_element_type=_f32))

    fs = fe[:, 0:1]
    fd = fe[:, 1:2]
    node_iota = lax.broadcasted_iota(jnp.int32, (_FEE, _FN), 1)
    S = (node_iota == fs).astype(_f32)
    Dm = (node_iota == fd).astype(_f32)
    ones_col = jnp.ones((_FEE, 1), _f32)
    cnt = lax.dot_general(Dm, ones_col, (((0,), (0,)), ((), ())),
                          preferred_element_type=_f32)
    cnt = jnp.maximum(cnt, 1.0)

    def sage(xin, wl, bl, wr):
        gath = jnp.dot(S, xin, preferred_element_type=_f32)
        sums = lax.dot_general(Dm, gath, (((0,), (0,)), ((), ())),
                               preferred_element_type=_f32)
        mean = sums / cnt
        out = jnp.dot(mean, wl[...], preferred_element_type=_f32)
        out += jnp.dot(xin, wr[...], preferred_element_type=_f32)
        return jnp.maximum(out + bl[...], 0.0)

    g = sage(xt, w1l, b1l, w1r)
    g = sage(g, w2l, b2l, w2r)

    bin_of = lax.broadcasted_iota(jnp.int32, (_B, _FN), 1) // _NPF
    bid = lax.broadcasted_iota(jnp.int32, (_B, _FN), 0)
    P = jnp.where(bin_of == bid, 1.0 / _NPF, 0.0)
    pooled = jnp.dot(P, g, preferred_element_type=_f32)

    z = jnp.dot(pooled, p1w[...], preferred_element_type=_f32) + p1b[...]
    z = jnp.dot(z, p2w[...], preferred_element_type=_f32) + p2b[...]
    z = jnp.dot(z, p3w[...], preferred_element_type=_f32) + p3b[...]
    o[...] = jax.nn.sigmoid(z)


def _fcg_tc(psum, pcnt, xext, fe, w1l, b1l, w1r, w2l, b2l, w2r,
            p1w, p1b, p2w, p2b, p3w, p3b):
    return pl.pallas_call(
        _fcg_body,
        out_shape=jax.ShapeDtypeStruct((_B, 1), _f32),
    )(psum, pcnt, xext, fe, w1l, b1l, w1r, w2l, b2l, w2r,
      p1w, p1b, p2w, p2b, p3w, p3b)


@jax.jit
def kernel(x, edge_index, cfg_batch, ext_names, func_edges,
           cfg1_Wl, cfg1_bl, cfg1_Wr, cfg2_Wl, cfg2_bl, cfg2_Wr,
           fcg1_Wl, fcg1_bl, fcg1_Wr, fcg2_Wl, fcg2_bl, fcg2_Wr,
           emb, pj1_W, pj1_b, pj2_W, pj2_b, pj3_W, pj3_b):
    src = edge_index[0]
    dst = edge_index[1]
    z128 = jnp.zeros((_WB, _D), _f32)
    z16 = jnp.zeros((_WB, 16), _f32)
    o16 = jnp.ones((_CH, 16), _f32)
    z8 = jnp.zeros((_ZC, 8), _f32)
    o8 = jnp.ones((_CH, 8), _f32)

    gat_call = _make_gat_call(True)
    gat2_call = _make_gat_call(False)
    sca_call = _make_sca_call()
    pool_call = _make_pool_call()
    dshift = (dst[None, :]
              + (jnp.arange(_NS, dtype=jnp.int32) * _N)[:, None]).reshape(-1)

    def _ungroup(aggT):
        u = aggT.reshape(_NC, _NS, _N, 8).transpose(0, 2, 1, 3)
        u = u.reshape(_NC, _N, _D)
        return u[0], u[1]

    g1, cnt3 = gat_call(x, src, dshift, z8, o8)
    ct = cnt3.reshape(_NC * _NS, _N, 8).transpose(1, 0, 2)
    ct = ct.reshape(_N, _NC * _NS * 8)
    a0, a1 = _ungroup(sca_call(g1, dshift, z8))
    h = _dense_tc(a0, a1, ct, x, cfg1_Wl, cfg1_bl.reshape(1, _D), cfg1_Wr)
    g2 = gat2_call(h, src, dshift, z8, o8)
    b0, b1 = _ungroup(sca_call(g2, dshift, z8))
    h2 = _dense_tc(b0, b1, ct, h, cfg2_Wl, cfg2_bl.reshape(1, _D), cfg2_Wr)

    psum, pcnt, xext = pool_call(h2, cfg_batch, ext_names.reshape(-1), emb,
                                 z128, z16, o16)

    off = (jnp.arange(_B, dtype=func_edges.dtype) * _NPF)[:, None, None]
    fe = (func_edges + off).transpose(1, 0, 2).reshape(2, _FEE).T
    fe = fe.astype(jnp.int32)

    return _fcg_tc(psum, pcnt, xext, fe,
                   fcg1_Wl, fcg1_bl.reshape(1, _D), fcg1_Wr,
                   fcg2_Wl, fcg2_bl.reshape(1, _D), fcg2_Wr,
                   pj1_W, pj1_b.reshape(1, -1), pj2_W, pj2_b.reshape(1, -1),
                   pj3_W, pj3_b.reshape(1, 1))


# speedup vs baseline: 1.7538x; 1.0042x over previous
import functools
import jax
import jax.numpy as jnp
from jax import lax
from jax.experimental import pallas as pl
from jax.experimental.pallas import tpu as pltpu
from jax.experimental.pallas import tpu_sc as plsc

_N = 10000
_E = 320000
_D = 128
_F = 256
_B = 8
_NE = 16
_FE = 256
_NPF = 48
_FN = _B * _NPF
_FEE = _B * _FE

_NC = 2
_NS = 16
_CH = 80
_EPW = _E // (_NC * _NS)
_NCHUNK = _EPW // _CH
_WB = 80
_NWB = _N // _WB
_WBPT = (_NWB + _NS - 1) // _NS

_f32 = jnp.float32


_CA = 128
_NCA = (_E // _NC) // _CA
_ZC = 80
_NZ = _N // _ZC


def _gat_core(with_cnt, feat, srci, dshift, z8, o8,
              gout, cnt_out,
              cacc, srcv0, srcv1, dstv, rows0, rows1, onev, zbuf,
              semg0, semg1, semw0, semw1, semc):
    c = lax.axis_index("c")
    s = lax.axis_index("s")
    w = c * _NS + s
    my0 = s * _N
    ebase = w * _EPW
    srcvs, rowss, semgs, semws = (srcv0, srcv1), (rows0, rows1), \
        (semg0, semg1), (semw0, semw1)

    if with_cnt:
        pltpu.sync_copy(z8, zbuf)

        @pl.loop(0, _NZ)
        def _zero(j):
            pltpu.sync_copy(zbuf, cacc.at[pl.ds(my0 + j * _ZC, _ZC)])

        pltpu.sync_copy(o8, onev)

    pltpu.sync_copy(srci.at[pl.ds(ebase, _CH)], srcv0)
    pltpu.async_copy(feat.at[srcv0], rows0, semg0)

    @pl.loop(0, _NCHUNK)
    def _chunk(i):
        base = pl.multiple_of(ebase + i * _CH, _CH)

        def _steps(b):
            sv, rw, sg, swr = srcvs[b], rowss[b], semgs[b], semws[b]
            pltpu.make_async_copy(feat.at[sv], rw, sg).wait()

            @pl.when(i + 1 < _NCHUNK)
            def _pf():
                nb = pl.multiple_of(base + _CH, _CH)
                osv, orw = srcvs[1 - b], rowss[1 - b]

                @pl.when(i >= 1)
                def _dw():
                    for g in range(_NS):
                        pltpu.make_async_copy(
                            orw.at[:, pl.ds(8 * g, 8)],
                            gout.at[pl.ds(g * _E + nb, _CH)],
                            semws[1 - b]).wait()

                pltpu.sync_copy(srci.at[pl.ds(nb, _CH)], osv)
                pltpu.async_copy(feat.at[osv], orw, semgs[1 - b])

            for g in range(_NS):
                pltpu.async_copy(rw.at[:, pl.ds(8 * g, 8)],
                                 gout.at[pl.ds(g * _E + base, _CH)], swr)

            if with_cnt:
                pltpu.sync_copy(dshift.at[pl.ds(s * _E + base, _CH)], dstv)
                pltpu.async_copy(onev, cacc.at[dstv], semc, add=True).wait()

        @pl.when(i % 2 == 0)
        def _b0():
            _steps(0)

        @pl.when(i % 2 == 1)
        def _b1():
            _steps(1)

    for b in range(2):
        last = ebase
        for g in range(_NS):
            pltpu.make_async_copy(rowss[b].at[:, pl.ds(8 * g, 8)],
                                  gout.at[pl.ds(g * _E + last, _CH)],
                                  semws[b]).wait()

    if with_cnt:
        @pl.loop(0, _NZ)
        def _wb(j):
            r0 = my0 + j * _ZC
            pltpu.sync_copy(cacc.at[pl.ds(r0, _ZC)], zbuf)
            pltpu.sync_copy(zbuf, cnt_out.at[pl.ds(c * _NS * _N + r0, _ZC)])


def _make_gat_call():
    mesh = plsc.VectorSubcoreMesh(core_axis_name="c", subcore_axis_name="s")

    def body(feat, srci, dshift, z8, o8, gout, cnt_out, *scr):
        _gat_core(True, feat, srci, dshift, z8, o8, gout, cnt_out, *scr)

    return pl.kernel(
        body,
        out_type=[
            jax.ShapeDtypeStruct((_NS * _E, 8), _f32),
            jax.ShapeDtypeStruct((_NC * _NS * _N, 8), _f32),
        ],
        mesh=mesh,
        scratch_types=[
            pltpu.VMEM_SHARED((_NS * _N, 8), _f32),
            pltpu.VMEM((_CH,), jnp.int32),
            pltpu.VMEM((_CH,), jnp.int32),
            pltpu.VMEM((_CH,), jnp.int32),
            pltpu.VMEM((_CH, _D), _f32),
            pltpu.VMEM((_CH, _D), _f32),
            pltpu.VMEM((_CH, 8), _f32),
            pltpu.VMEM((_ZC, 8), _f32),
            pltpu.SemaphoreType.DMA,
            pltpu.SemaphoreType.DMA,
            pltpu.SemaphoreType.DMA,
            pltpu.SemaphoreType.DMA,
            pltpu.SemaphoreType.DMA,
        ],
        compiler_params=pltpu.CompilerParams(use_tc_tiling_on_sc=False),
    )


_SG = 320
_EPT2 = 10240


def _gat2_body(feat, srci, gout,
               srcv0, srcv1, rows0, rows1, semg0, semg1, semw0, semw1):
    c = lax.axis_index("c")
    s = lax.axis_index("s")
    ebase = c * (_E // _NC) + s * _EPT2
    nsc = jnp.where(s == _NS - 1, (_E // _NC - 15 * _EPT2) // _SG, _EPT2 // _SG)
    srcvs, rowss = (srcv0, srcv1), (rows0, rows1)
    semgs, semws = (semg0, semg1), (semw0, semw1)

    def _issue(i, b):
        base = pl.multiple_of(ebase + i * _SG, _SG)
        pltpu.sync_copy(srci.at[pl.ds(base, _SG)], srcvs[b])
        for q in range(_SG // _CH):
            pltpu.async_copy(
                feat.at[srcvs[b].at[pl.ds(q * _CH, _CH)]],
                rowss[b].at[pl.ds(q * _CH, _CH)], semgs[b])

    _issue(0, 0)

    @pl.loop(0, _EPT2 // _SG // 2)
    def _grp(g):
        for b in range(2):
            i = g * 2 + b

            @pl.when(i < nsc)
            def _do():
                base = pl.multiple_of(ebase + i * _SG, _SG)
                @pl.when(i + 1 < nsc)
                def _pf():
                    @pl.when(i >= 1)
                    def _dw():
                        for gg in range(_NS):
                            pltpu.make_async_copy(
                                rowss[1 - b].at[:, pl.ds(8 * gg, 8)],
                                gout.at[pl.ds(gg * _E + base, _SG)],
                                semws[1 - b]).wait()

                    _issue(i + 1, 1 - b)

                for q in range(_SG // _CH):
                    pltpu.make_async_copy(
                        feat.at[srcvs[b].at[pl.ds(q * _CH, _CH)]],
                        rowss[b].at[pl.ds(q * _CH, _CH)], semgs[b]).wait()
                for gg in range(_NS):
                    pltpu.async_copy(rowss[b].at[:, pl.ds(8 * gg, 8)],
                                     gout.at[pl.ds(gg * _E + base, _SG)],
                                     semws[b])

    for b in range(2):
        for gg in range(_NS):
            pltpu.make_async_copy(rowss[b].at[:, pl.ds(8 * gg, 8)],
                                  gout.at[pl.ds(gg * _E + ebase, _SG)],
                                  semws[b]).wait()


def _make_gat2_call():
    mesh = plsc.VectorSubcoreMesh(core_axis_name="c", subcore_axis_name="s")
    return pl.kernel(
        _gat2_body,
        out_type=jax.ShapeDtypeStruct((_NS * _E, 8), _f32),
        mesh=mesh,
        scratch_types=[
            pltpu.VMEM((_SG,), jnp.int32),
            pltpu.VMEM((_SG,), jnp.int32),
            pltpu.VMEM((_SG, _D), _f32),
            pltpu.VMEM((_SG, _D), _f32),
            pltpu.SemaphoreType.DMA,
            pltpu.SemaphoreType.DMA,
            pltpu.SemaphoreType.DMA,
            pltpu.SemaphoreType.DMA,
        ],
        compiler_params=pltpu.CompilerParams(use_tc_tiling_on_sc=False),
    )


def _sca_body(gout, dshift, z8,
              agg_out,
              acc, dstv0, dstv1, rows0, rows1, zbuf,
              seml0, seml1, sems0, sems1):
    c = lax.axis_index("c")
    s = lax.axis_index("s")
    my0 = s * _N
    dstvs, rowss = (dstv0, dstv1), (rows0, rows1)
    semls, semss = (seml0, seml1), (sems0, sems1)

    pltpu.sync_copy(z8, zbuf)

    @pl.loop(0, _NZ)
    def _zero(j):
        pltpu.sync_copy(zbuf, acc.at[pl.ds(my0 + j * _ZC, _ZC)])

    ebase = c * (_E // _NC)
    ibase = s * _E + ebase

    def _loads(i, b):
        off = pl.multiple_of(i * _CA, _CA)
        pltpu.async_copy(dshift.at[pl.ds(ibase + off, _CA)], dstvs[b],
                         semls[b])
        pltpu.async_copy(gout.at[pl.ds(ibase + off, _CA)], rowss[b],
                         semls[b])

    _loads(0, 0)
    _loads(1, 1)

    @pl.loop(0, _NCA // 2)
    def _grp(g):
        i0 = g * 2
        for b in range(2):
            i = i0 + b
            off = pl.multiple_of(i * _CA, _CA)
            pltpu.make_async_copy(dshift.at[pl.ds(ibase + off, _CA)],
                                  dstvs[b], semls[b]).wait()
            pltpu.make_async_copy(gout.at[pl.ds(ibase + off, _CA)],
                                  rowss[b], semls[b]).wait()
            pltpu.async_copy(rowss[b], acc.at[dstvs[b]], semss[b], add=True)
        for b in range(2):
            pltpu.make_async_copy(rowss[b], acc.at[dstvs[b]],
                                  semss[b]).wait()

            @pl.when(g + 1 < _NCA // 2)
            def _rf():
                _loads(i0 + 2 + b, b)

    @pl.loop(0, _NZ)
    def _wb(j):
        r0 = my0 + j * _ZC
        pltpu.sync_copy(acc.at[pl.ds(r0, _ZC)], zbuf)
        pltpu.sync_copy(zbuf, agg_out.at[pl.ds(c * _NS * _N + r0, _ZC)])


def _make_sca_call():
    mesh = plsc.VectorSubcoreMesh(core_axis_name="c", subcore_axis_name="s")
    return pl.kernel(
        _sca_body,
        out_type=jax.ShapeDtypeStruct((_NC * _NS * _N, 8), _f32),
        mesh=mesh,
        scratch_types=[
            pltpu.VMEM_SHARED((_NS * _N, 8), _f32),
            pltpu.VMEM((_CA,), jnp.int32),
            pltpu.VMEM((_CA,), jnp.int32),
            pltpu.VMEM((_CA, 8), _f32),
            pltpu.VMEM((_CA, 8), _f32),
            pltpu.VMEM((_ZC, 8), _f32),
            pltpu.SemaphoreType.DMA,
            pltpu.SemaphoreType.DMA,
            pltpu.SemaphoreType.DMA,
            pltpu.SemaphoreType.DMA,
        ],
        compiler_params=pltpu.CompilerParams(use_tc_tiling_on_sc=False),
    )


def _dense_tc_body(a0, a1, ct, x, wl, bl, wr, o):
    ones = jnp.full((_NC * _NS * 8, 1), 0.125, _f32)
    cnt = jnp.maximum(jnp.dot(ct[...], ones, preferred_element_type=_f32), 1.0)
    mean = (a0[...] + a1[...]) / cnt
    acc = jnp.dot(mean, wl[...], preferred_element_type=_f32)
    acc += jnp.dot(x[...], wr[...], preferred_element_type=_f32)
    o[...] = jnp.maximum(acc + bl[...], 0.0)


def _dense_tc(a0, a1, ct, x, wl, bl, wr):
    R = 1000
    grid = (_N // R,)
    row = lambda i: (i, 0)
    return pl.pallas_call(
        _dense_tc_body,
        grid=grid,
        in_specs=[
            pl.BlockSpec((R, _D), row),
            pl.BlockSpec((R, _D), row),
            pl.BlockSpec((R, _NC * _NS * 8), row),
            pl.BlockSpec((R, _D), row),
            pl.BlockSpec((_D, _D), lambda i: (0, 0)),
            pl.BlockSpec((1, _D), lambda i: (0, 0)),
            pl.BlockSpec((_D, _D), lambda i: (0, 0)),
        ],
        out_specs=pl.BlockSpec((R, _D), row),
        out_shape=jax.ShapeDtypeStruct((_N, _D), _f32),
    )(a0, a1, ct, x, wl, bl, wr)


def _pool_body(h, seg, extids, emb, z128, z16, o16,
               psum_out, pcnt_out, ext_out,
               psum, pcnt, segv, rows, onev, sbuf, cbuf, xbuf, cacc,
               idv, erows, sem):
    c = lax.axis_index("c")
    s = lax.axis_index("s")

    @pl.when(c == 0)
    def _zero():
        pltpu.sync_copy(z128.at[pl.ds(0, 64)], rows.at[pl.ds(0, 64)])
        for j in range(_F // 64):
            pltpu.sync_copy(rows.at[pl.ds(0, 64)],
                            psum.at[pl.ds(s * _F + j * 64, 64)])
        pltpu.sync_copy(z16.at[pl.ds(0, 64)], onev.at[pl.ds(0, 64)])
        for j in range(_F // 64):
            pltpu.sync_copy(onev.at[pl.ds(0, 64)],
                            pcnt.at[pl.ds(s * _F + j * 64, 64)])
        pltpu.sync_copy(o16, onev)

    @pl.when(c == 1)
    def _emb():
        pltpu.sync_copy(extids.at[pl.ds(s * 8, 8)], idv)
        pltpu.async_copy(emb.at[idv], erows, sem).wait()
        pltpu.sync_copy(erows, ext_out.at[pl.ds(s * 8, 8)])

    plsc.subcore_barrier()

    @pl.when(c == 0)
    def _pool():
        @pl.loop(0, (_NCHUNK + _NS - 1) // _NS)
        def _j(j):
            chunk = s + _NS * j

            @pl.when(chunk < _NCHUNK)
            def _do():
                base = pl.multiple_of(chunk * _CH, _CH)
                pltpu.sync_copy(seg.at[pl.ds(base, _CH)], segv)
                for k in range(_CH // 16):
                    segv[pl.ds(k * 16, 16)] = (
                        segv[pl.ds(k * 16, 16)] + s * _F)
                pltpu.sync_copy(h.at[pl.ds(base, _CH)], rows)
                pltpu.sync_copy(rows, psum.at[segv], add=True)
                pltpu.sync_copy(onev, pcnt.at[segv], add=True)

    plsc.subcore_barrier()

    @pl.when(c == 0)
    def _wb():
        pltpu.sync_copy(psum.at[pl.ds(s * 16, 16)], xbuf)
        pltpu.sync_copy(pcnt.at[pl.ds(s * 16, 16)], cacc)

        @pl.loop(1, _NS)
        def _t(t):
            pltpu.sync_copy(psum.at[pl.ds(t * _F + s * 16, 16)], sbuf)
            pltpu.sync_copy(pcnt.at[pl.ds(t * _F + s * 16, 16)], cbuf)
            for r in range(16):
                for k in range(_D // 16):
                    xbuf[r, pl.ds(k * 16, 16)] = (
                        xbuf[r, pl.ds(k * 16, 16)]
                        + sbuf[r, pl.ds(k * 16, 16)])
                cacc[r, :] = cacc[r, :] + cbuf[r, :]

        pltpu.sync_copy(xbuf, psum_out.at[pl.ds(s * 16, 16)])
        pltpu.sync_copy(cacc, pcnt_out.at[pl.ds(s * 16, 16)])


def _make_pool_call():
    mesh = plsc.VectorSubcoreMesh(core_axis_name="c", subcore_axis_name="s")
    return pl.kernel(
        _pool_body,
        out_type=[
            jax.ShapeDtypeStruct((_F, _D), _f32),
            jax.ShapeDtypeStruct((_F, 16), _f32),
            jax.ShapeDtypeStruct((_B * _NE, _D), _f32),
        ],
        mesh=mesh,
        scratch_types=[
            pltpu.VMEM_SHARED((_NS * _F, _D), _f32),
            pltpu.VMEM_SHARED((_NS * _F, 16), _f32),
            pltpu.VMEM((_CH,), jnp.int32),
            pltpu.VMEM((_CH, _D), _f32),
            pltpu.VMEM((_CH, 16), _f32),
            pltpu.VMEM((16, _D), _f32),
            pltpu.VMEM((16, 16), _f32),
            pltpu.VMEM((16, _D), _f32),
            pltpu.VMEM((16, 16), _f32),
            pltpu.VMEM((8,), jnp.int32),
            pltpu.VMEM((8, _D), _f32),
            pltpu.SemaphoreType.DMA,
        ],
    )


def _fcg_body(psum, pcnt, xext, fe, w1l, b1l, w1r, w2l, b2l, w2r,
              p1w, p1b, p2w, p2b, p3w, p3b, o):
    xcfg = psum[...] / jnp.maximum(pcnt[:, 0:1], 1.0)
    r_b = lax.broadcasted_iota(jnp.int32, (_FN, _F), 0) // _NPF
    r_i = lax.broadcasted_iota(jnp.int32, (_FN, _F), 0) % _NPF
    q = lax.broadcasted_iota(jnp.int32, (_FN, _F), 1)
    A1 = jnp.where((r_i < _F // _B) & (q == r_b * (_F // _B) + r_i), 1.0, 0.0)
    r_b2 = lax.broadcasted_iota(jnp.int32, (_FN, _B * _NE), 0) // _NPF
    r_i2 = lax.broadcasted_iota(jnp.int32, (_FN, _B * _NE), 0) % _NPF
    p = lax.broadcasted_iota(jnp.int32, (_FN, _B * _NE), 1)
    A2 = jnp.where((r_i2 >= _F // _B) & (p == r_b2 * _NE + r_i2 - _F // _B),
                   1.0, 0.0)
    xt = (jnp.dot(A1, xcfg, preferred_element_type=_f32)
          + jnp.dot(A2, xext[...], preferred_element_type=_f32))

    fs = fe[:, 0:1]
    fd = fe[:, 1:2]
    node_iota = lax.broadcasted_iota(jnp.int32, (_FEE, _FN), 1)
    S = (node_iota == fs).astype(_f32)
    Dm = (node_iota == fd).astype(_f32)
    ones_col = jnp.ones((_FEE, 1), _f32)
    cnt = lax.dot_general(Dm, ones_col, (((0,), (0,)), ((), ())),
                          preferred_element_type=_f32)
    cnt = jnp.maximum(cnt, 1.0)

    def sage(xin, wl, bl, wr):
        gath = jnp.dot(S, xin, preferred_element_type=_f32)
        sums = lax.dot_general(Dm, gath, (((0,), (0,)), ((), ())),
                               preferred_element_type=_f32)
        mean = sums / cnt
        out = jnp.dot(mean, wl[...], preferred_element_type=_f32)
        out += jnp.dot(xin, wr[...], preferred_element_type=_f32)
        return jnp.maximum(out + bl[...], 0.0)

    g = sage(xt, w1l, b1l, w1r)
    g = sage(g, w2l, b2l, w2r)

    bin_of = lax.broadcasted_iota(jnp.int32, (_B, _FN), 1) // _NPF
    bid = lax.broadcasted_iota(jnp.int32, (_B, _FN), 0)
    P = jnp.where(bin_of == bid, 1.0 / _NPF, 0.0)
    pooled = jnp.dot(P, g, preferred_element_type=_f32)

    z = jnp.dot(pooled, p1w[...], preferred_element_type=_f32) + p1b[...]
    z = jnp.dot(z, p2w[...], preferred_element_type=_f32) + p2b[...]
    z = jnp.dot(z, p3w[...], preferred_element_type=_f32) + p3b[...]
    o[...] = jax.nn.sigmoid(z)


def _fcg_tc(psum, pcnt, xext, fe, w1l, b1l, w1r, w2l, b2l, w2r,
            p1w, p1b, p2w, p2b, p3w, p3b):
    return pl.pallas_call(
        _fcg_body,
        out_shape=jax.ShapeDtypeStruct((_B, 1), _f32),
    )(psum, pcnt, xext, fe, w1l, b1l, w1r, w2l, b2l, w2r,
      p1w, p1b, p2w, p2b, p3w, p3b)


@jax.jit
def kernel(x, edge_index, cfg_batch, ext_names, func_edges,
           cfg1_Wl, cfg1_bl, cfg1_Wr, cfg2_Wl, cfg2_bl, cfg2_Wr,
           fcg1_Wl, fcg1_bl, fcg1_Wr, fcg2_Wl, fcg2_bl, fcg2_Wr,
           emb, pj1_W, pj1_b, pj2_W, pj2_b, pj3_W, pj3_b):
    src = edge_index[0]
    dst = edge_index[1]
    z128 = jnp.zeros((_WB, _D), _f32)
    z16 = jnp.zeros((_WB, 16), _f32)
    o16 = jnp.ones((_CH, 16), _f32)
    z8 = jnp.zeros((_ZC, 8), _f32)
    o8 = jnp.ones((_CH, 8), _f32)

    gat_call = _make_gat_call()
    gat2_call = _make_gat2_call()
    sca_call = _make_sca_call()
    pool_call = _make_pool_call()
    dshift = (dst[None, :]
              + (jnp.arange(_NS, dtype=jnp.int32) * _N)[:, None]).reshape(-1)

    def _ungroup(aggT):
        u = aggT.reshape(_NC, _NS, _N, 8).transpose(0, 2, 1, 3)
        u = u.reshape(_NC, _N, _D)
        return u[0], u[1]

    g1, cnt3 = gat_call(x, src, dshift, z8, o8)
    ct = cnt3.reshape(_NC * _NS, _N, 8).transpose(1, 0, 2)
    ct = ct.reshape(_N, _NC * _NS * 8)
    a0, a1 = _ungroup(sca_call(g1, dshift, z8))
    h = _dense_tc(a0, a1, ct, x, cfg1_Wl, cfg1_bl.reshape(1, _D), cfg1_Wr)
    g2 = gat2_call(h, src)
    b0, b1 = _ungroup(sca_call(g2, dshift, z8))
    h2 = _dense_tc(b0, b1, ct, h, cfg2_Wl, cfg2_bl.reshape(1, _D), cfg2_Wr)

    psum, pcnt, xext = pool_call(h2, cfg_batch, ext_names.reshape(-1), emb,
                                 z128, z16, o16)

    off = (jnp.arange(_B, dtype=func_edges.dtype) * _NPF)[:, None, None]
    fe = (func_edges + off).transpose(1, 0, 2).reshape(2, _FEE).T
    fe = fe.astype(jnp.int32)

    return _fcg_tc(psum, pcnt, xext, fe,
                   fcg1_Wl, fcg1_bl.reshape(1, _D), fcg1_Wr,
                   fcg2_Wl, fcg2_bl.reshape(1, _D), fcg2_Wr,
                   pj1_W, pj1_b.reshape(1, -1), pj2_W, pj2_b.reshape(1, -1),
                   pj3_W, pj3_b.reshape(1, 1))


# speedup vs baseline: 1.8489x; 1.0543x over previous
import functools
import jax
import jax.numpy as jnp
from jax import lax
from jax.experimental import pallas as pl
from jax.experimental.pallas import tpu as pltpu
from jax.experimental.pallas import tpu_sc as plsc

_N = 10000
_E = 320000
_D = 128
_F = 256
_B = 8
_NE = 16
_FE = 256
_NPF = 48
_FN = _B * _NPF
_FEE = _B * _FE

_NC = 2
_NS = 16
_CH = 80
_EPW = _E // (_NC * _NS)
_NCHUNK = _EPW // _CH
_WB = 80
_NWB = _N // _WB
_WBPT = (_NWB + _NS - 1) // _NS

_f32 = jnp.float32


_CA = 128
_NCA = (_E // _NC) // _CA
_ZC = 80
_NZ = _N // _ZC


def _gat_core(with_cnt, feat, srci, dshift, z8, o8,
              gout, cnt_out,
              cacc, srcv0, srcv1, dstv, rows0, rows1, onev, zbuf,
              semg0, semg1, semw0, semw1, semc):
    c = lax.axis_index("c")
    s = lax.axis_index("s")
    w = c * _NS + s
    my0 = s * _N
    ebase = w * _EPW
    srcvs, rowss, semgs, semws = (srcv0, srcv1), (rows0, rows1), \
        (semg0, semg1), (semw0, semw1)

    if with_cnt:
        pltpu.sync_copy(z8, zbuf)

        @pl.loop(0, _NZ)
        def _zero(j):
            pltpu.sync_copy(zbuf, cacc.at[pl.ds(my0 + j * _ZC, _ZC)])

        pltpu.sync_copy(o8, onev)

    pltpu.sync_copy(srci.at[pl.ds(ebase, _CH)], srcv0)
    pltpu.async_copy(feat.at[srcv0], rows0, semg0)

    @pl.loop(0, _NCHUNK)
    def _chunk(i):
        base = pl.multiple_of(ebase + i * _CH, _CH)

        def _steps(b):
            sv, rw, sg, swr = srcvs[b], rowss[b], semgs[b], semws[b]
            pltpu.make_async_copy(feat.at[sv], rw, sg).wait()

            @pl.when(i + 1 < _NCHUNK)
            def _pf():
                nb = pl.multiple_of(base + _CH, _CH)
                osv, orw = srcvs[1 - b], rowss[1 - b]

                @pl.when(i >= 1)
                def _dw():
                    for g in range(_NS):
                        pltpu.make_async_copy(
                            orw.at[:, pl.ds(8 * g, 8)],
                            gout.at[pl.ds(g * _E + nb, _CH)],
                            semws[1 - b]).wait()

                pltpu.sync_copy(srci.at[pl.ds(nb, _CH)], osv)
                pltpu.async_copy(feat.at[osv], orw, semgs[1 - b])

            for g in range(_NS):
                pltpu.async_copy(rw.at[:, pl.ds(8 * g, 8)],
                                 gout.at[pl.ds(g * _E + base, _CH)], swr)

            if with_cnt:
                pltpu.sync_copy(dshift.at[pl.ds(s * _E + base, _CH)], dstv)
                pltpu.async_copy(onev, cacc.at[dstv], semc, add=True).wait()

        @pl.when(i % 2 == 0)
        def _b0():
            _steps(0)

        @pl.when(i % 2 == 1)
        def _b1():
            _steps(1)

    for b in range(2):
        last = ebase
        for g in range(_NS):
            pltpu.make_async_copy(rowss[b].at[:, pl.ds(8 * g, 8)],
                                  gout.at[pl.ds(g * _E + last, _CH)],
                                  semws[b]).wait()

    if with_cnt:
        @pl.loop(0, _NZ)
        def _wb(j):
            r0 = my0 + j * _ZC
            pltpu.sync_copy(cacc.at[pl.ds(r0, _ZC)], zbuf)
            pltpu.sync_copy(zbuf, cnt_out.at[pl.ds(c * _NS * _N + r0, _ZC)])


def _make_gat_call():
    mesh = plsc.VectorSubcoreMesh(core_axis_name="c", subcore_axis_name="s")

    def body(feat, srci, dshift, z8, o8, gout, cnt_out, *scr):
        _gat_core(True, feat, srci, dshift, z8, o8, gout, cnt_out, *scr)

    return pl.kernel(
        body,
        out_type=[
            jax.ShapeDtypeStruct((_NS * _E, 8), _f32),
            jax.ShapeDtypeStruct((_NC * _NS * _N, 8), _f32),
        ],
        mesh=mesh,
        scratch_types=[
            pltpu.VMEM_SHARED((_NS * _N, 8), _f32),
            pltpu.VMEM((_CH,), jnp.int32),
            pltpu.VMEM((_CH,), jnp.int32),
            pltpu.VMEM((_CH,), jnp.int32),
            pltpu.VMEM((_CH, _D), _f32),
            pltpu.VMEM((_CH, _D), _f32),
            pltpu.VMEM((_CH, 8), _f32),
            pltpu.VMEM((_ZC, 8), _f32),
            pltpu.SemaphoreType.DMA,
            pltpu.SemaphoreType.DMA,
            pltpu.SemaphoreType.DMA,
            pltpu.SemaphoreType.DMA,
            pltpu.SemaphoreType.DMA,
        ],
        compiler_params=pltpu.CompilerParams(use_tc_tiling_on_sc=False),
    )


_SG = 320
_EPT2 = 10240


def _gat2_body(feat, srci, gout,
               srcv0, srcv1, rows0, rows1, semg0, semg1, semw0, semw1):
    c = lax.axis_index("c")
    s = lax.axis_index("s")
    ebase = c * (_E // _NC) + s * _EPT2
    nsc = jnp.where(s == _NS - 1, (_E // _NC - 15 * _EPT2) // _SG, _EPT2 // _SG)
    srcvs, rowss = (srcv0, srcv1), (rows0, rows1)
    semgs, semws = (semg0, semg1), (semw0, semw1)

    def _issue(i, b):
        base = pl.multiple_of(ebase + i * _SG, _SG)
        pltpu.sync_copy(srci.at[pl.ds(base, _SG)], srcvs[b])
        for q in range(_SG // _CH):
            pltpu.async_copy(
                feat.at[srcvs[b].at[pl.ds(q * _CH, _CH)]],
                rowss[b].at[pl.ds(q * _CH, _CH)], semgs[b])

    _issue(0, 0)

    @pl.loop(0, _EPT2 // _SG // 2)
    def _grp(g):
        for b in range(2):
            i = g * 2 + b

            @pl.when(i < nsc)
            def _do():
                base = pl.multiple_of(ebase + i * _SG, _SG)
                @pl.when(i + 1 < nsc)
                def _pf():
                    @pl.when(i >= 1)
                    def _dw():
                        for gg in range(_NS):
                            pltpu.make_async_copy(
                                rowss[1 - b].at[:, pl.ds(8 * gg, 8)],
                                gout.at[pl.ds(gg * _E + base, _SG)],
                                semws[1 - b]).wait()

                    _issue(i + 1, 1 - b)

                for q in range(_SG // _CH):
                    pltpu.make_async_copy(
                        feat.at[srcvs[b].at[pl.ds(q * _CH, _CH)]],
                        rowss[b].at[pl.ds(q * _CH, _CH)], semgs[b]).wait()
                for gg in range(_NS):
                    pltpu.async_copy(rowss[b].at[:, pl.ds(8 * gg, 8)],
                                     gout.at[pl.ds(gg * _E + base, _SG)],
                                     semws[b])

    for b in range(2):
        for gg in range(_NS):
            pltpu.make_async_copy(rowss[b].at[:, pl.ds(8 * gg, 8)],
                                  gout.at[pl.ds(gg * _E + ebase, _SG)],
                                  semws[b]).wait()


def _make_gat2_call():
    mesh = plsc.VectorSubcoreMesh(core_axis_name="c", subcore_axis_name="s")
    return pl.kernel(
        _gat2_body,
        out_type=jax.ShapeDtypeStruct((_NS * _E, 8), _f32),
        mesh=mesh,
        scratch_types=[
            pltpu.VMEM((_SG,), jnp.int32),
            pltpu.VMEM((_SG,), jnp.int32),
            pltpu.VMEM((_SG, _D), _f32),
            pltpu.VMEM((_SG, _D), _f32),
            pltpu.SemaphoreType.DMA,
            pltpu.SemaphoreType.DMA,
            pltpu.SemaphoreType.DMA,
            pltpu.SemaphoreType.DMA,
        ],
        compiler_params=pltpu.CompilerParams(use_tc_tiling_on_sc=False),
    )


def _sca_body(gout, dshift, z8,
              agg_out,
              acc, dstv0, dstv1, rows0, rows1, zbuf,
              seml0, seml1, sems0, sems1):
    c = lax.axis_index("c")
    s = lax.axis_index("s")
    my0 = s * _N
    dstvs, rowss = (dstv0, dstv1), (rows0, rows1)
    semls, semss = (seml0, seml1), (sems0, sems1)

    pltpu.sync_copy(z8, zbuf)

    @pl.loop(0, _NZ)
    def _zero(j):
        pltpu.sync_copy(zbuf, acc.at[pl.ds(my0 + j * _ZC, _ZC)])

    ebase = c * (_E // _NC)
    ibase = s * _E + ebase

    def _loads(i, b):
        off = pl.multiple_of(i * _CA, _CA)
        pltpu.async_copy(dshift.at[pl.ds(ibase + off, _CA)], dstvs[b],
                         semls[b])
        pltpu.async_copy(gout.at[pl.ds(ibase + off, _CA)], rowss[b],
                         semls[b])

    _loads(0, 0)
    _loads(1, 1)

    @pl.loop(0, _NCA // 2)
    def _grp(g):
        i0 = g * 2
        for b in range(2):
            i = i0 + b
            off = pl.multiple_of(i * _CA, _CA)
            pltpu.make_async_copy(dshift.at[pl.ds(ibase + off, _CA)],
                                  dstvs[b], semls[b]).wait()
            pltpu.make_async_copy(gout.at[pl.ds(ibase + off, _CA)],
                                  rowss[b], semls[b]).wait()
            pltpu.async_copy(rowss[b], acc.at[dstvs[b]], semss[b], add=True)
        for b in range(2):
            pltpu.make_async_copy(rowss[b], acc.at[dstvs[b]],
                                  semss[b]).wait()

            @pl.when(g + 1 < _NCA // 2)
            def _rf():
                _loads(i0 + 2 + b, b)

    @pl.loop(0, _NZ)
    def _wb(j):
        pltpu.sync_copy(acc.at[pl.ds(my0 + j * _ZC, _ZC)], zbuf)
        pltpu.sync_copy(zbuf, agg_out.at[pl.ds(c * _N + j * _ZC, _ZC),
                                         pl.ds(8 * s, 8)])


def _make_sca_call():
    mesh = plsc.VectorSubcoreMesh(core_axis_name="c", subcore_axis_name="s")
    return pl.kernel(
        _sca_body,
        out_type=jax.ShapeDtypeStruct((_NC * _N, _D), _f32),
        mesh=mesh,
        scratch_types=[
            pltpu.VMEM_SHARED((_NS * _N, 8), _f32),
            pltpu.VMEM((_CA,), jnp.int32),
            pltpu.VMEM((_CA,), jnp.int32),
            pltpu.VMEM((_CA, 8), _f32),
            pltpu.VMEM((_CA, 8), _f32),
            pltpu.VMEM((_ZC, 8), _f32),
            pltpu.SemaphoreType.DMA,
            pltpu.SemaphoreType.DMA,
            pltpu.SemaphoreType.DMA,
            pltpu.SemaphoreType.DMA,
        ],
        compiler_params=pltpu.CompilerParams(use_tc_tiling_on_sc=False),
    )


def _dense_tc_body(a0, a1, ct, x, wl, bl, wr, o):
    ones = jnp.full((_NC * _NS * 8, 1), 0.125, _f32)
    cnt = jnp.maximum(jnp.dot(ct[...], ones, preferred_element_type=_f32), 1.0)
    mean = (a0[...] + a1[...]) / cnt
    acc = jnp.dot(mean, wl[...], preferred_element_type=_f32)
    acc += jnp.dot(x[...], wr[...], preferred_element_type=_f32)
    o[...] = jnp.maximum(acc + bl[...], 0.0)


def _dense_tc(a0, a1, ct, x, wl, bl, wr):
    R = 1000
    grid = (_N // R,)
    row = lambda i: (i, 0)
    return pl.pallas_call(
        _dense_tc_body,
        grid=grid,
        in_specs=[
            pl.BlockSpec((R, _D), row),
            pl.BlockSpec((R, _D), row),
            pl.BlockSpec((R, _NC * _NS * 8), row),
            pl.BlockSpec((R, _D), row),
            pl.BlockSpec((_D, _D), lambda i: (0, 0)),
            pl.BlockSpec((1, _D), lambda i: (0, 0)),
            pl.BlockSpec((_D, _D), lambda i: (0, 0)),
        ],
        out_specs=pl.BlockSpec((R, _D), row),
        out_shape=jax.ShapeDtypeStruct((_N, _D), _f32),
    )(a0, a1, ct, x, wl, bl, wr)


def _pool_body(h, seg, extids, emb, z128, z16, o16,
               psum_out, pcnt_out, ext_out,
               psum, pcnt, segv, rows, onev, sbuf, cbuf, xbuf, cacc,
               idv, erows, sem):
    c = lax.axis_index("c")
    s = lax.axis_index("s")

    @pl.when(c == 0)
    def _zero():
        pltpu.sync_copy(z128.at[pl.ds(0, 64)], rows.at[pl.ds(0, 64)])
        for j in range(_F // 64):
            pltpu.sync_copy(rows.at[pl.ds(0, 64)],
                            psum.at[pl.ds(s * _F + j * 64, 64)])
        pltpu.sync_copy(z16.at[pl.ds(0, 64)], onev.at[pl.ds(0, 64)])
        for j in range(_F // 64):
            pltpu.sync_copy(onev.at[pl.ds(0, 64)],
                            pcnt.at[pl.ds(s * _F + j * 64, 64)])
        pltpu.sync_copy(o16, onev)

    @pl.when(c == 1)
    def _emb():
        pltpu.sync_copy(extids.at[pl.ds(s * 8, 8)], idv)
        pltpu.async_copy(emb.at[idv], erows, sem).wait()
        pltpu.sync_copy(erows, ext_out.at[pl.ds(s * 8, 8)])

    plsc.subcore_barrier()

    @pl.when(c == 0)
    def _pool():
        @pl.loop(0, (_NCHUNK + _NS - 1) // _NS)
        def _j(j):
            chunk = s + _NS * j

            @pl.when(chunk < _NCHUNK)
            def _do():
                base = pl.multiple_of(chunk * _CH, _CH)
                pltpu.sync_copy(seg.at[pl.ds(base, _CH)], segv)
                for k in range(_CH // 16):
                    segv[pl.ds(k * 16, 16)] = (
                        segv[pl.ds(k * 16, 16)] + s * _F)
                pltpu.sync_copy(h.at[pl.ds(base, _CH)], rows)
                pltpu.sync_copy(rows, psum.at[segv], add=True)
                pltpu.sync_copy(onev, pcnt.at[segv], add=True)

    plsc.subcore_barrier()

    @pl.when(c == 0)
    def _wb():
        pltpu.sync_copy(psum.at[pl.ds(s * 16, 16)], xbuf)
        pltpu.sync_copy(pcnt.at[pl.ds(s * 16, 16)], cacc)

        @pl.loop(1, _NS)
        def _t(t):
            pltpu.sync_copy(psum.at[pl.ds(t * _F + s * 16, 16)], sbuf)
            pltpu.sync_copy(pcnt.at[pl.ds(t * _F + s * 16, 16)], cbuf)
            for r in range(16):
                for k in range(_D // 16):
                    xbuf[r, pl.ds(k * 16, 16)] = (
                        xbuf[r, pl.ds(k * 16, 16)]
                        + sbuf[r, pl.ds(k * 16, 16)])
                cacc[r, :] = cacc[r, :] + cbuf[r, :]

        pltpu.sync_copy(xbuf, psum_out.at[pl.ds(s * 16, 16)])
        pltpu.sync_copy(cacc, pcnt_out.at[pl.ds(s * 16, 16)])


def _make_pool_call():
    mesh = plsc.VectorSubcoreMesh(core_axis_name="c", subcore_axis_name="s")
    return pl.kernel(
        _pool_body,
        out_type=[
            jax.ShapeDtypeStruct((_F, _D), _f32),
            jax.ShapeDtypeStruct((_F, 16), _f32),
            jax.ShapeDtypeStruct((_B * _NE, _D), _f32),
        ],
        mesh=mesh,
        scratch_types=[
            pltpu.VMEM_SHARED((_NS * _F, _D), _f32),
            pltpu.VMEM_SHARED((_NS * _F, 16), _f32),
            pltpu.VMEM((_CH,), jnp.int32),
            pltpu.VMEM((_CH, _D), _f32),
            pltpu.VMEM((_CH, 16), _f32),
            pltpu.VMEM((16, _D), _f32),
            pltpu.VMEM((16, 16), _f32),
            pltpu.VMEM((16, _D), _f32),
            pltpu.VMEM((16, 16), _f32),
            pltpu.VMEM((8,), jnp.int32),
            pltpu.VMEM((8, _D), _f32),
            pltpu.SemaphoreType.DMA,
        ],
    )


def _fcg_body(psum, pcnt, xext, fe, w1l, b1l, w1r, w2l, b2l, w2r,
              p1w, p1b, p2w, p2b, p3w, p3b, o):
    xcfg = psum[...] / jnp.maximum(pcnt[:, 0:1], 1.0)
    r_b = lax.broadcasted_iota(jnp.int32, (_FN, _F), 0) // _NPF
    r_i = lax.broadcasted_iota(jnp.int32, (_FN, _F), 0) % _NPF
    q = lax.broadcasted_iota(jnp.int32, (_FN, _F), 1)
    A1 = jnp.where((r_i < _F // _B) & (q == r_b * (_F // _B) + r_i), 1.0, 0.0)
    r_b2 = lax.broadcasted_iota(jnp.int32, (_FN, _B * _NE), 0) // _NPF
    r_i2 = lax.broadcasted_iota(jnp.int32, (_FN, _B * _NE), 0) % _NPF
    p = lax.broadcasted_iota(jnp.int32, (_FN, _B * _NE), 1)
    A2 = jnp.where((r_i2 >= _F // _B) & (p == r_b2 * _NE + r_i2 - _F // _B),
                   1.0, 0.0)
    xt = (jnp.dot(A1, xcfg, preferred_element_type=_f32)
          + jnp.dot(A2, xext[...], preferred_element_type=_f32))

    fs = fe[:, 0:1]
    fd = fe[:, 1:2]
    node_iota = lax.broadcasted_iota(jnp.int32, (_FEE, _FN), 1)
    S = (node_iota == fs).astype(_f32)
    Dm = (node_iota == fd).astype(_f32)
    ones_col = jnp.ones((_FEE, 1), _f32)
    cnt = lax.dot_general(Dm, ones_col, (((0,), (0,)), ((), ())),
                          preferred_element_type=_f32)
    cnt = jnp.maximum(cnt, 1.0)

    def sage(xin, wl, bl, wr):
        gath = jnp.dot(S, xin, preferred_element_type=_f32)
        sums = lax.dot_general(Dm, gath, (((0,), (0,)), ((), ())),
                               preferred_element_type=_f32)
        mean = sums / cnt
        out = jnp.dot(mean, wl[...], preferred_element_type=_f32)
        out += jnp.dot(xin, wr[...], preferred_element_type=_f32)
        return jnp.maximum(out + bl[...], 0.0)

    g = sage(xt, w1l, b1l, w1r)
    g = sage(g, w2l, b2l, w2r)

    bin_of = lax.broadcasted_iota(jnp.int32, (_B, _FN), 1) // _NPF
    bid = lax.broadcasted_iota(jnp.int32, (_B, _FN), 0)
    P = jnp.where(bin_of == bid, 1.0 / _NPF, 0.0)
    pooled = jnp.dot(P, g, preferred_element_type=_f32)

    z = jnp.dot(pooled, p1w[...], preferred_element_type=_f32) + p1b[...]
    z = jnp.dot(z, p2w[...], preferred_element_type=_f32) + p2b[...]
    z = jnp.dot(z, p3w[...], preferred_element_type=_f32) + p3b[...]
    o[...] = jax.nn.sigmoid(z)


def _fcg_tc(psum, pcnt, xext, fe, w1l, b1l, w1r, w2l, b2l, w2r,
            p1w, p1b, p2w, p2b, p3w, p3b):
    return pl.pallas_call(
        _fcg_body,
        out_shape=jax.ShapeDtypeStruct((_B, 1), _f32),
    )(psum, pcnt, xext, fe, w1l, b1l, w1r, w2l, b2l, w2r,
      p1w, p1b, p2w, p2b, p3w, p3b)


@jax.jit
def kernel(x, edge_index, cfg_batch, ext_names, func_edges,
           cfg1_Wl, cfg1_bl, cfg1_Wr, cfg2_Wl, cfg2_bl, cfg2_Wr,
           fcg1_Wl, fcg1_bl, fcg1_Wr, fcg2_Wl, fcg2_bl, fcg2_Wr,
           emb, pj1_W, pj1_b, pj2_W, pj2_b, pj3_W, pj3_b):
    src = edge_index[0]
    dst = edge_index[1]
    z128 = jnp.zeros((_WB, _D), _f32)
    z16 = jnp.zeros((_WB, 16), _f32)
    o16 = jnp.ones((_CH, 16), _f32)
    z8 = jnp.zeros((_ZC, 8), _f32)
    o8 = jnp.ones((_CH, 8), _f32)

    gat_call = _make_gat_call()
    gat2_call = _make_gat2_call()
    sca_call = _make_sca_call()
    pool_call = _make_pool_call()
    dshift = (dst[None, :]
              + (jnp.arange(_NS, dtype=jnp.int32) * _N)[:, None]).reshape(-1)

    g1, cnt3 = gat_call(x, src, dshift, z8, o8)
    ct = cnt3.reshape(_NC * _NS, _N, 8).transpose(1, 0, 2)
    ct = ct.reshape(_N, _NC * _NS * 8)
    agg1 = sca_call(g1, dshift, z8)
    h = _dense_tc(agg1[:_N], agg1[_N:], ct, x,
                  cfg1_Wl, cfg1_bl.reshape(1, _D), cfg1_Wr)
    g2 = gat2_call(h, src)
    agg2 = sca_call(g2, dshift, z8)
    h2 = _dense_tc(agg2[:_N], agg2[_N:], ct, h,
                   cfg2_Wl, cfg2_bl.reshape(1, _D), cfg2_Wr)

    psum, pcnt, xext = pool_call(h2, cfg_batch, ext_names.reshape(-1), emb,
                                 z128, z16, o16)

    off = (jnp.arange(_B, dtype=func_edges.dtype) * _NPF)[:, None, None]
    fe = (func_edges + off).transpose(1, 0, 2).reshape(2, _FEE).T
    fe = fe.astype(jnp.int32)

    return _fcg_tc(psum, pcnt, xext, fe,
                   fcg1_Wl, fcg1_bl.reshape(1, _D), fcg1_Wr,
                   fcg2_Wl, fcg2_bl.reshape(1, _D), fcg2_Wr,
                   pj1_W, pj1_b.reshape(1, -1), pj2_W, pj2_b.reshape(1, -1),
                   pj3_W, pj3_b.reshape(1, 1))
